# baseline (speedup 1.0000x reference)
import functools

import jax
import jax.numpy as jnp
from jax import lax
from jax.experimental import pallas as pl
from jax.experimental.pallas import tpu as pltpu
from jax.experimental.pallas import tpu_sc as plsc

N = 320
NUM_NODES = N * N
NOUT = NUM_NODES * 4
NW = 32
ROWS_W = N // NW
VALS_W = ROWS_W * N * 4
SLAB_ROWS = 24
VPR = (N * 4) // 16

_MESH = plsc.VectorSubcoreMesh(core_axis_name="c", subcore_axis_name="s",
                               num_cores=2, num_subcores=16)


@functools.partial(
    pl.kernel,
    out_type=jax.ShapeDtypeStruct((NOUT,), jnp.float32),
    mesh=_MESH,
    compiler_params=pltpu.CompilerParams(needs_layout_passes=False),
    scratch_types=(
        pltpu.VMEM((SLAB_ROWS, N), jnp.float32),
        pltpu.VMEM((VALS_W,), jnp.float32),
        pltpu.SemaphoreType.DMA,
    ),
)
def _values_sc(grid_hbm, vals_hbm, slab_v, vals_v, sem):
    wid = lax.axis_index("s") * 2 + lax.axis_index("c")
    i0 = wid * ROWS_W
    s = pl.multiple_of(
        jnp.clip(((i0 - 1) // 8) * 8, 0, N - SLAB_ROWS), 8)

    pltpu.sync_copy(grid_hbm.at[pl.ds(s, SLAB_ROWS)], slab_v)

    lane = lax.iota(jnp.int32, 16)
    k = lane & 3
    l4 = lane >> 2
    di = jnp.where(k == 0, 1, jnp.where(k == 1, -1, 0))
    dj = jnp.where(k == 2, 1, jnp.where(k == 3, -1, 0))

    copies = []
    for r in range(ROWS_W):
        i = i0 + r
        ti = i + di
        ok_i = (ti >= 0) & (ti < N)
        ti_loc = jnp.clip(ti, 0, N - 1) - s

        @plsc.parallel_loop(0, VPR, unroll=4)
        def vec_body(jj):
            jv = jj * 4 + l4
            tj = jv + dj
            m = ok_i & (tj >= 0) & (tj < N)
            val = plsc.load_gather(slab_v, [ti_loc, jnp.clip(tj, 0, N - 1)])
            vals_v[pl.ds(r * (N * 4) + jj * 16, 16)] = jnp.where(m, val, 0.0)

        copies.append(pltpu.async_copy(
            vals_v.at[pl.ds(r * (N * 4), N * 4)],
            vals_hbm.at[pl.ds(wid * VALS_W + r * (N * 4), N * 4)], sem))

    for c in copies:
        c.wait()


def _rc_tc_body(rows_ref, cols_ref):
    idx = lax.broadcasted_iota(jnp.int32, (NOUT,), 0)
    k = idx & 3
    v = idx >> 2
    i = v // N
    j = v - i * N
    ti = i + jnp.where(k == 0, 1, jnp.where(k == 1, -1, 0))
    tj = j + jnp.where(k == 2, 1, jnp.where(k == 3, -1, 0))
    m = (ti >= 0) & (ti < N) & (tj >= 0) & (tj < N)
    rows_ref[...] = jnp.where(m, v, 0)
    cols_ref[...] = jnp.where(m, ti * N + tj, 0)


_rc_tc = pl.pallas_call(
    _rc_tc_body,
    out_shape=(
        jax.ShapeDtypeStruct((NOUT,), jnp.int32),
        jax.ShapeDtypeStruct((NOUT,), jnp.int32),
    ),
)


def kernel(grid):
    vals = _values_sc(grid)
    rows, cols = _rc_tc()
    return vals, jnp.stack([rows, cols], axis=1)


# speedup vs baseline: 1.1917x; 1.1917x over previous
import functools

import jax
import jax.numpy as jnp
from jax import lax
from jax.experimental import pallas as pl
from jax.experimental.pallas import tpu as pltpu
from jax.experimental.pallas import tpu_sc as plsc

N = 320
NUM_NODES = N * N
NOUT = NUM_NODES * 4
NW = 32
ROWS_W = N // NW
VALS_W = ROWS_W * N * 4
SLAB_ROWS = 24
VPR = (N * 4) // 16

_MESH = plsc.VectorSubcoreMesh(core_axis_name="c", subcore_axis_name="s",
                               num_cores=2, num_subcores=16)


@functools.partial(
    pl.kernel,
    out_type=jax.ShapeDtypeStruct((NOUT,), jnp.float32),
    mesh=_MESH,
    compiler_params=pltpu.CompilerParams(needs_layout_passes=False),
    scratch_types=(
        pltpu.VMEM((SLAB_ROWS, N), jnp.float32),
        pltpu.VMEM((VALS_W,), jnp.float32),
        pltpu.SemaphoreType.DMA,
    ),
)
def _values_sc(grid_hbm, vals_hbm, slab_v, vals_v, sem):
    wid = lax.axis_index("s") * 2 + lax.axis_index("c")
    i0 = wid * ROWS_W
    s = pl.multiple_of(
        jnp.clip(((i0 - 1) // 8) * 8, 0, N - SLAB_ROWS), 8)

    pltpu.sync_copy(grid_hbm.at[pl.ds(s, SLAB_ROWS)], slab_v)

    lane = lax.iota(jnp.int32, 16)
    k = lane & 3
    l4 = lane >> 2
    di = jnp.where(k == 0, 1, jnp.where(k == 1, -1, 0))
    dj = jnp.where(k == 2, 1, jnp.where(k == 3, -1, 0))

    copies = []
    for r in range(ROWS_W):
        i = i0 + r
        ti = i + di
        ok_i = (ti >= 0) & (ti < N)
        ti_loc = jnp.clip(ti, 0, N - 1) - s

        @plsc.parallel_loop(0, VPR, unroll=4)
        def vec_body(jj):
            jv = jj * 4 + l4
            tj = jv + dj
            m = ok_i & (tj >= 0) & (tj < N)
            val = plsc.load_gather(slab_v, [ti_loc, jnp.clip(tj, 0, N - 1)])
            vals_v[pl.ds(r * (N * 4) + jj * 16, 16)] = jnp.where(m, val, 0.0)

        copies.append(pltpu.async_copy(
            vals_v.at[pl.ds(r * (N * 4), N * 4)],
            vals_hbm.at[pl.ds(wid * VALS_W + r * (N * 4), N * 4)], sem))

    for c in copies:
        c.wait()


def _rc_tc_body(rows_ref, cols_ref):
    idx = lax.broadcasted_iota(jnp.int32, (NOUT,), 0)
    k = idx & 3
    v = idx >> 2
    i = ((v >> 6) * 52429) >> 18
    j = v - i * N
    ti = i + jnp.where(k == 0, 1, jnp.where(k == 1, -1, 0))
    tj = j + jnp.where(k == 2, 1, jnp.where(k == 3, -1, 0))
    m = (ti >= 0) & (ti < N) & (tj >= 0) & (tj < N)
    rows_ref[...] = jnp.where(m, v, 0)
    cols_ref[...] = jnp.where(m, ti * N + tj, 0)


_rc_tc = pl.pallas_call(
    _rc_tc_body,
    out_shape=(
        jax.ShapeDtypeStruct((NOUT,), jnp.int32),
        jax.ShapeDtypeStruct((NOUT,), jnp.int32),
    ),
)


def kernel(grid):
    vals = _values_sc(grid)
    rows, cols = _rc_tc()
    return vals, jnp.stack([rows, cols], axis=1)


# speedup vs baseline: 1.7058x; 1.4314x over previous
import functools

import jax
import jax.numpy as jnp
from jax import lax
from jax.experimental import pallas as pl
from jax.experimental.pallas import tpu as pltpu
from jax.experimental.pallas import tpu_sc as plsc

N = 320
NUM_NODES = N * N
NOUT = NUM_NODES * 4
NW = 32
ROWS_W = N // NW
VALS_W = ROWS_W * N * 4
SLAB_ROWS = 24
VPR = (N * 4) // 16

_MESH = plsc.VectorSubcoreMesh(core_axis_name="c", subcore_axis_name="s",
                               num_cores=2, num_subcores=16)


@functools.partial(
    pl.kernel,
    out_type=jax.ShapeDtypeStruct((NOUT,), jnp.float32),
    mesh=_MESH,
    compiler_params=pltpu.CompilerParams(needs_layout_passes=False),
    scratch_types=(
        pltpu.VMEM((SLAB_ROWS, N), jnp.float32),
        pltpu.VMEM((VALS_W,), jnp.float32),
        pltpu.SemaphoreType.DMA,
    ),
)
def _values_sc(grid_hbm, vals_hbm, slab_v, vals_v, sem):
    wid = lax.axis_index("s") * 2 + lax.axis_index("c")
    i0 = wid * ROWS_W
    s = pl.multiple_of(
        jnp.clip(((i0 - 1) // 8) * 8, 0, N - SLAB_ROWS), 8)

    pltpu.sync_copy(grid_hbm.at[pl.ds(s, SLAB_ROWS)], slab_v)

    lane = lax.iota(jnp.int32, 16)
    k = lane & 3
    l4 = lane >> 2
    di = jnp.where(k == 0, 1, jnp.where(k == 1, -1, 0))
    dj = jnp.where(k == 2, 1, jnp.where(k == 3, -1, 0))

    copies = []
    for r in range(ROWS_W):
        i = i0 + r
        ti = i + di
        ok_i = (ti >= 0) & (ti < N)
        ti_loc = jnp.clip(ti, 0, N - 1) - s

        @plsc.parallel_loop(0, VPR, unroll=4)
        def vec_body(jj):
            jv = jj * 4 + l4
            tj = jv + dj
            m = ok_i & (tj >= 0) & (tj < N)
            val = plsc.load_gather(slab_v, [ti_loc, jnp.clip(tj, 0, N - 1)])
            vals_v[pl.ds(r * (N * 4) + jj * 16, 16)] = jnp.where(m, val, 0.0)

        copies.append(pltpu.async_copy(
            vals_v.at[pl.ds(r * (N * 4), N * 4)],
            vals_hbm.at[pl.ds(wid * VALS_W + r * (N * 4), N * 4)], sem))

    for c in copies:
        c.wait()


_RC_GRID = 8
_RC_ROWS = NOUT // 128 // _RC_GRID


def _rc_tc_body(rows_ref, cols_ref):
    g = pl.program_id(0)
    grow = (g * _RC_ROWS
            + lax.broadcasted_iota(jnp.int32, (_RC_ROWS, 128), 0))
    idx = grow * 128 + lax.broadcasted_iota(jnp.int32, (_RC_ROWS, 128), 1)
    k = idx & 3
    v = idx >> 2
    i = ((v >> 6) * 52429) >> 18
    j = v - i * N
    ti = i + jnp.where(k == 0, 1, jnp.where(k == 1, -1, 0))
    tj = j + jnp.where(k == 2, 1, jnp.where(k == 3, -1, 0))
    m = (ti >= 0) & (ti < N) & (tj >= 0) & (tj < N)
    rows_ref[...] = jnp.where(m, v, 0)
    cols_ref[...] = jnp.where(m, ti * N + tj, 0)


_rc_tc = pl.pallas_call(
    _rc_tc_body,
    grid=(_RC_GRID,),
    out_specs=(
        pl.BlockSpec((_RC_ROWS, 128), lambda g: (g, 0)),
        pl.BlockSpec((_RC_ROWS, 128), lambda g: (g, 0)),
    ),
    out_shape=(
        jax.ShapeDtypeStruct((NOUT // 128, 128), jnp.int32),
        jax.ShapeDtypeStruct((NOUT // 128, 128), jnp.int32),
    ),
)


def kernel(grid):
    vals = _values_sc(grid)
    rows, cols = _rc_tc()
    return vals, jnp.stack([rows.reshape(NOUT), cols.reshape(NOUT)], axis=1)
